# baseline (speedup 1.0000x reference)
import functools

import jax
import jax.numpy as jnp
from jax import lax
from jax.experimental import pallas as pl
from jax.experimental.pallas import tpu as pltpu
from jax.experimental.pallas import tpu_sc as plsc

D = 2048
S = 100
NUM_LEVELS = 100
B = 512
C = 26

B_SC = 160
B_TC = B - B_SC

LANES = 16
BLANES = 32
SP = 112
NW = 32
BPW = B_SC // NW
DC = 128
NCHUNK = D // DC
TPB = DC // BLANES
G = 8

LP = 128
DT = 512


def _sc_encode(xt_pad, id_bf, lvl_bf):
  mesh = plsc.VectorSubcoreMesh(core_axis_name="c", subcore_axis_name="s")

  @functools.partial(
      pl.kernel,
      out_type=jax.ShapeDtypeStruct((NW, BPW, D), jnp.bfloat16),
      mesh=mesh,
      compiler_params=pltpu.CompilerParams(use_tc_tiling_on_sc=False),
      scratch_types=[
          pltpu.VMEM((SP, BPW), jnp.float32),
          pltpu.VMEM((S, BPW), jnp.int32),
          pltpu.VMEM((2, NUM_LEVELS, DC), jnp.bfloat16),
          pltpu.VMEM((2, S, DC), jnp.bfloat16),
          pltpu.VMEM((2, BPW, DC), jnp.bfloat16),
          pltpu.SemaphoreType.DMA,
          pltpu.SemaphoreType.DMA,
          pltpu.SemaphoreType.DMA,
          pltpu.SemaphoreType.DMA,
          pltpu.SemaphoreType.DMA,
          pltpu.SemaphoreType.DMA,
      ],
  )
  def enc_kernel(xt_hbm, id_hbm, lvl_hbm, out_hbm, xt_v, idx_v, lvl_v, id_v,
                 acc_v, s_lvl0, s_lvl1, s_id0, s_id1, s_out0, s_out1):
    wid = lax.axis_index("s") * 2 + lax.axis_index("c")
    s_lvl = [s_lvl0, s_lvl1]
    s_id = [s_id0, s_id1]
    s_out = [s_out0, s_out1]

    def start_in(ci, slot):
      col = ci * DC
      pltpu.async_copy(lvl_hbm.at[:, pl.ds(col, DC)], lvl_v.at[slot],
                       s_lvl[slot])
      pltpu.async_copy(id_hbm.at[:, pl.ds(col, DC)], id_v.at[slot],
                       s_id[slot])

    def wait_in(slot):
      pltpu.make_async_copy(lvl_hbm.at[:, pl.ds(0, DC)], lvl_v.at[slot],
                            s_lvl[slot]).wait()
      pltpu.make_async_copy(id_hbm.at[:, pl.ds(0, DC)], id_v.at[slot],
                            s_id[slot]).wait()

    def start_out(ci, slot):
      col = ci * DC
      pltpu.async_copy(acc_v.at[slot], out_hbm.at[wid, :, pl.ds(col, DC)],
                       s_out[slot])

    def wait_out(slot):
      pltpu.make_async_copy(acc_v.at[slot],
                            out_hbm.at[wid, :, pl.ds(0, DC)],
                            s_out[slot]).wait()

    start_in(0, 0)
    pltpu.sync_copy(xt_hbm.at[wid], xt_v)

    def idx_body(s, _):
      v = xt_v[s, :] * jnp.float32(NUM_LEVELS - 1) + jnp.float32(0.5)
      iv = v.astype(jnp.int32)
      iv = jnp.minimum(jnp.maximum(iv, 0), NUM_LEVELS - 1)
      idx_v[s, :] = iv
      return 0

    lax.fori_loop(0, S, idx_body, 0)

    def compute(slot, pi):
      lvl_s = lvl_v.at[slot]
      id_s = id_v.at[slot]
      acc_s = acc_v.at[slot]

      @pl.when(pi > 0)
      def _():
        wait_out(slot)

      zero = jnp.zeros((BLANES,), jnp.bfloat16)

      def zero_body(b, _):
        for t in range(TPB):
          acc_s[b, pl.ds(t * BLANES, BLANES)] = zero
        return 0

      lax.fori_loop(0, BPW, zero_body, 0)

      def g_core(s0, glen):
        vidx = [idx_v[s0 + g, :] for g in range(glen)]
        idrow = [
            [id_s[s0 + g, pl.ds(t * BLANES, BLANES)] for t in range(TPB)]
            for g in range(glen)
        ]
        for b in range(BPW):
          ib = [vidx[g][b] for g in range(glen)]
          accs = [acc_s[b, pl.ds(t * BLANES, BLANES)] for t in range(TPB)]
          for g in range(glen):
            lr = [lvl_s[ib[g], pl.ds(t * BLANES, BLANES)] for t in range(TPB)]
            for t in range(TPB):
              accs[t] = accs[t] + lr[t] * idrow[g][t]
          for t in range(TPB):
            acc_s[b, pl.ds(t * BLANES, BLANES)] = accs[t]

      def g_body(gi, _):
        g_core(gi * G, G)
        return 0

      lax.fori_loop(0, (S // G), g_body, 0)
      if S % G:
        g_core(S - S % G, S % G)

      one = jnp.float32(1.0).astype(jnp.bfloat16)
      mone = jnp.float32(-1.0).astype(jnp.bfloat16)
      zbf = jnp.float32(0.0).astype(jnp.bfloat16)

      def sign_body(b, _):
        for t in range(TPB):
          a = acc_s[b, pl.ds(t * BLANES, BLANES)]
          acc_s[b, pl.ds(t * BLANES, BLANES)] = jnp.where(a > zbf, one, mone)
        return 0

      lax.fori_loop(0, BPW, sign_body, 0)

    def pair_body(pi, _):
      c0 = pi * 2
      wait_in(0)
      start_in(c0 + 1, 1)
      compute(0, pi)
      start_out(c0, 0)
      wait_in(1)

      @pl.when(pi < NCHUNK // 2 - 1)
      def _():
        start_in(c0 + 2, 0)

      compute(1, pi)
      start_out(c0 + 1, 1)
      return 0

    lax.fori_loop(0, NCHUNK // 2, pair_body, 0)
    wait_out(0)
    wait_out(1)

  return enc_kernel(xt_pad, id_bf, lvl_bf)


def _tc_encode_classify(x_tc, id_bf, lvl_pad, wt_pad):
  cp = wt_pad.shape[1]

  def body(x_ref, id_ref, lvl_ref, w_ref, out_ref):
    idxf = x_ref[:] * jnp.float32(NUM_LEVELS - 1) + jnp.float32(0.5)
    idxi = jnp.clip(idxf.astype(jnp.int32), 0, NUM_LEVELS - 1)
    lcol = lax.broadcasted_iota(jnp.int32, (B_TC, LP), 1)
    lvl = lvl_ref[:]
    acc = jnp.zeros((B_TC, DT), jnp.float32)
    for s in range(S):
      oh = (idxi[:, s:s + 1] == lcol).astype(jnp.bfloat16)
      ts = lvl * id_ref[s:s + 1, :].astype(jnp.bfloat16)
      acc = acc + jnp.dot(oh, ts, preferred_element_type=jnp.float32)
    enc = jnp.where(acc > 0, jnp.float32(1.0), jnp.float32(-1.0))
    part = jnp.dot(enc, w_ref[:], preferred_element_type=jnp.float32)
    i = pl.program_id(0)

    @pl.when(i == 0)
    def _():
      out_ref[:] = part

    @pl.when(i != 0)
    def _():
      out_ref[:] = out_ref[:] + part

  return pl.pallas_call(
      body,
      grid=(D // DT,),
      in_specs=[
          pl.BlockSpec((B_TC, S), lambda i: (0, 0)),
          pl.BlockSpec((S, DT), lambda i: (0, i)),
          pl.BlockSpec((LP, DT), lambda i: (0, i)),
          pl.BlockSpec((DT, cp), lambda i: (i, 0)),
      ],
      out_specs=pl.BlockSpec((B_TC, cp), lambda i: (0, 0)),
      out_shape=jax.ShapeDtypeStruct((B_TC, cp), jnp.float32),
  )(x_tc, id_bf, lvl_pad, wt_pad)


def _tc_classify(enc, wt_pad):
  cp = wt_pad.shape[1]

  def body(enc_ref, w_ref, out_ref):
    out_ref[:] = jnp.dot(enc_ref[:].astype(jnp.float32), w_ref[:],
                         preferred_element_type=jnp.float32)

  return pl.pallas_call(
      body,
      out_shape=jax.ShapeDtypeStruct((enc.shape[0], cp), jnp.float32),
  )(enc, wt_pad)


@jax.jit
def kernel(x, id_weight, level_weight, classify_weight):
  id_bf = id_weight.astype(jnp.bfloat16)
  lvl_bf = level_weight.astype(jnp.bfloat16)
  xt_pad = jnp.zeros((SP, B_SC), jnp.float32).at[:S].set(x[:B_SC].T)
  xt3 = xt_pad.reshape(SP, NW, BPW).transpose(1, 0, 2)
  lvl_pad = jnp.zeros((LP, D), jnp.bfloat16).at[:NUM_LEVELS].set(lvl_bf)
  cp = 128
  wt_pad = jnp.zeros((D, cp), jnp.float32).at[:, :C].set(classify_weight.T)
  enc_sc = _sc_encode(xt3, id_bf, lvl_bf).reshape(B_SC, D)
  logit_tc = _tc_encode_classify(x[B_SC:], id_bf, lvl_pad, wt_pad)
  logit_sc = _tc_classify(enc_sc, wt_pad)
  return jnp.concatenate([logit_sc[:, :C], logit_tc[:, :C]], axis=0)


# speedup vs baseline: 1.0410x; 1.0410x over previous
import functools

import jax
import jax.numpy as jnp
from jax import lax
from jax.experimental import pallas as pl
from jax.experimental.pallas import tpu as pltpu
from jax.experimental.pallas import tpu_sc as plsc

D = 2048
S = 100
NUM_LEVELS = 100
B = 512
C = 26

B_SC = 192
B_TC = B - B_SC

LANES = 16
BLANES = 32
SP = 112
NW = 32
BPW = B_SC // NW
DC = 128
NCHUNK = D // DC
TPB = DC // BLANES
G = 8

LP = 128
DT = 512


def _sc_encode(xt_pad, id_bf, lvl_bf):
  mesh = plsc.VectorSubcoreMesh(core_axis_name="c", subcore_axis_name="s")

  @functools.partial(
      pl.kernel,
      out_type=jax.ShapeDtypeStruct((NW, BPW, D), jnp.bfloat16),
      mesh=mesh,
      compiler_params=pltpu.CompilerParams(use_tc_tiling_on_sc=False),
      scratch_types=[
          pltpu.VMEM((SP, BPW), jnp.float32),
          pltpu.VMEM((S, BPW), jnp.int32),
          pltpu.VMEM((2, NUM_LEVELS, DC), jnp.bfloat16),
          pltpu.VMEM((2, S, DC), jnp.bfloat16),
          pltpu.VMEM((2, BPW, DC), jnp.bfloat16),
          pltpu.SemaphoreType.DMA,
          pltpu.SemaphoreType.DMA,
          pltpu.SemaphoreType.DMA,
          pltpu.SemaphoreType.DMA,
          pltpu.SemaphoreType.DMA,
          pltpu.SemaphoreType.DMA,
      ],
  )
  def enc_kernel(xt_hbm, id_hbm, lvl_hbm, out_hbm, xt_v, idx_v, lvl_v, id_v,
                 acc_v, s_lvl0, s_lvl1, s_id0, s_id1, s_out0, s_out1):
    wid = lax.axis_index("s") * 2 + lax.axis_index("c")
    s_lvl = [s_lvl0, s_lvl1]
    s_id = [s_id0, s_id1]
    s_out = [s_out0, s_out1]

    def start_in(ci, slot):
      col = ci * DC
      pltpu.async_copy(lvl_hbm.at[:, pl.ds(col, DC)], lvl_v.at[slot],
                       s_lvl[slot])
      pltpu.async_copy(id_hbm.at[:, pl.ds(col, DC)], id_v.at[slot],
                       s_id[slot])

    def wait_in(slot):
      pltpu.make_async_copy(lvl_hbm.at[:, pl.ds(0, DC)], lvl_v.at[slot],
                            s_lvl[slot]).wait()
      pltpu.make_async_copy(id_hbm.at[:, pl.ds(0, DC)], id_v.at[slot],
                            s_id[slot]).wait()

    def start_out(ci, slot):
      col = ci * DC
      pltpu.async_copy(acc_v.at[slot], out_hbm.at[wid, :, pl.ds(col, DC)],
                       s_out[slot])

    def wait_out(slot):
      pltpu.make_async_copy(acc_v.at[slot],
                            out_hbm.at[wid, :, pl.ds(0, DC)],
                            s_out[slot]).wait()

    start_in(0, 0)
    pltpu.sync_copy(xt_hbm.at[wid], xt_v)

    def idx_body(s, _):
      v = xt_v[s, :] * jnp.float32(NUM_LEVELS - 1) + jnp.float32(0.5)
      iv = v.astype(jnp.int32)
      iv = jnp.minimum(jnp.maximum(iv, 0), NUM_LEVELS - 1)
      idx_v[s, :] = iv
      return 0

    lax.fori_loop(0, S, idx_body, 0)

    def compute(slot, pi):
      lvl_s = lvl_v.at[slot]
      id_s = id_v.at[slot]
      acc_s = acc_v.at[slot]

      @pl.when(pi > 0)
      def _():
        wait_out(slot)

      zero = jnp.zeros((BLANES,), jnp.bfloat16)

      def zero_body(b, _):
        for t in range(TPB):
          acc_s[b, pl.ds(t * BLANES, BLANES)] = zero
        return 0

      lax.fori_loop(0, BPW, zero_body, 0)

      def g_core(s0, glen):
        vidx = [idx_v[s0 + g, :] for g in range(glen)]
        idrow = [
            [id_s[s0 + g, pl.ds(t * BLANES, BLANES)] for t in range(TPB)]
            for g in range(glen)
        ]
        for b in range(BPW):
          ib = [vidx[g][b] for g in range(glen)]
          accs = [acc_s[b, pl.ds(t * BLANES, BLANES)] for t in range(TPB)]
          for g in range(glen):
            lr = [lvl_s[ib[g], pl.ds(t * BLANES, BLANES)] for t in range(TPB)]
            for t in range(TPB):
              accs[t] = accs[t] + lr[t] * idrow[g][t]
          for t in range(TPB):
            acc_s[b, pl.ds(t * BLANES, BLANES)] = accs[t]

      def g_body(gi, _):
        g_core(gi * G, G)
        return 0

      lax.fori_loop(0, (S // G), g_body, 0)
      if S % G:
        g_core(S - S % G, S % G)

      one = jnp.float32(1.0).astype(jnp.bfloat16)
      mone = jnp.float32(-1.0).astype(jnp.bfloat16)
      zbf = jnp.float32(0.0).astype(jnp.bfloat16)

      def sign_body(b, _):
        for t in range(TPB):
          a = acc_s[b, pl.ds(t * BLANES, BLANES)]
          acc_s[b, pl.ds(t * BLANES, BLANES)] = jnp.where(a > zbf, one, mone)
        return 0

      lax.fori_loop(0, BPW, sign_body, 0)

    def pair_body(pi, _):
      c0 = pi * 2
      wait_in(0)
      start_in(c0 + 1, 1)
      compute(0, pi)
      start_out(c0, 0)
      wait_in(1)

      @pl.when(pi < NCHUNK // 2 - 1)
      def _():
        start_in(c0 + 2, 0)

      compute(1, pi)
      start_out(c0 + 1, 1)
      return 0

    lax.fori_loop(0, NCHUNK // 2, pair_body, 0)
    wait_out(0)
    wait_out(1)

  return enc_kernel(xt_pad, id_bf, lvl_bf)


def _tc_encode_classify(x_tc, id_bf, lvl_pad, wt_pad):
  cp = wt_pad.shape[1]

  def body(x_ref, id_ref, lvl_ref, w_ref, out_ref):
    idxf = x_ref[:] * jnp.float32(NUM_LEVELS - 1) + jnp.float32(0.5)
    idxi = jnp.clip(idxf.astype(jnp.int32), 0, NUM_LEVELS - 1)
    lcol = lax.broadcasted_iota(jnp.int32, (B_TC, LP), 1)
    lvl = lvl_ref[:]
    acc = jnp.zeros((B_TC, DT), jnp.float32)
    for s in range(S):
      oh = (idxi[:, s:s + 1] == lcol).astype(jnp.bfloat16)
      ts = lvl * id_ref[s:s + 1, :].astype(jnp.bfloat16)
      acc = acc + jnp.dot(oh, ts, preferred_element_type=jnp.float32)
    enc = jnp.where(acc > 0, jnp.float32(1.0), jnp.float32(-1.0))
    part = jnp.dot(enc, w_ref[:], preferred_element_type=jnp.float32)
    i = pl.program_id(0)

    @pl.when(i == 0)
    def _():
      out_ref[:] = part

    @pl.when(i != 0)
    def _():
      out_ref[:] = out_ref[:] + part

  return pl.pallas_call(
      body,
      grid=(D // DT,),
      in_specs=[
          pl.BlockSpec((B_TC, S), lambda i: (0, 0)),
          pl.BlockSpec((S, DT), lambda i: (0, i)),
          pl.BlockSpec((LP, DT), lambda i: (0, i)),
          pl.BlockSpec((DT, cp), lambda i: (i, 0)),
      ],
      out_specs=pl.BlockSpec((B_TC, cp), lambda i: (0, 0)),
      out_shape=jax.ShapeDtypeStruct((B_TC, cp), jnp.float32),
  )(x_tc, id_bf, lvl_pad, wt_pad)


def _tc_classify(enc, wt_pad):
  cp = wt_pad.shape[1]

  def body(enc_ref, w_ref, out_ref):
    out_ref[:] = jnp.dot(enc_ref[:].astype(jnp.float32), w_ref[:],
                         preferred_element_type=jnp.float32)

  return pl.pallas_call(
      body,
      out_shape=jax.ShapeDtypeStruct((enc.shape[0], cp), jnp.float32),
  )(enc, wt_pad)


@jax.jit
def kernel(x, id_weight, level_weight, classify_weight):
  id_bf = id_weight.astype(jnp.bfloat16)
  lvl_bf = level_weight.astype(jnp.bfloat16)
  xt_pad = jnp.zeros((SP, B_SC), jnp.float32).at[:S].set(x[:B_SC].T)
  xt3 = xt_pad.reshape(SP, NW, BPW).transpose(1, 0, 2)
  lvl_pad = jnp.zeros((LP, D), jnp.bfloat16).at[:NUM_LEVELS].set(lvl_bf)
  cp = 128
  wt_pad = jnp.zeros((D, cp), jnp.float32).at[:, :C].set(classify_weight.T)
  enc_sc = _sc_encode(xt3, id_bf, lvl_bf).reshape(B_SC, D)
  logit_tc = _tc_encode_classify(x[B_SC:], id_bf, lvl_pad, wt_pad)
  logit_sc = _tc_classify(enc_sc, wt_pad)
  return jnp.concatenate([logit_sc[:, :C], logit_tc[:, :C]], axis=0)


# speedup vs baseline: 1.0502x; 1.0089x over previous
import functools

import jax
import jax.numpy as jnp
from jax import lax
from jax.experimental import pallas as pl
from jax.experimental.pallas import tpu as pltpu
from jax.experimental.pallas import tpu_sc as plsc

D = 2048
S = 100
NUM_LEVELS = 100
B = 512
C = 26

B_SC = 192
B_TC = B - B_SC

LANES = 16
BLANES = 32
SP = 112
NW = 32
BPW = B_SC // NW
DC = 128
NCHUNK = D // DC
TPB = DC // BLANES
G = 8

LP = 128
DT = 512


def _sc_encode(xt_pad, id_bf, lvl_bf):
  mesh = plsc.VectorSubcoreMesh(core_axis_name="c", subcore_axis_name="s")

  @functools.partial(
      pl.kernel,
      out_type=jax.ShapeDtypeStruct((NW, BPW, D), jnp.bfloat16),
      mesh=mesh,
      compiler_params=pltpu.CompilerParams(use_tc_tiling_on_sc=False),
      scratch_types=[
          pltpu.VMEM((SP, LANES), jnp.float32),
          pltpu.VMEM((S, LANES), jnp.int32),
          pltpu.VMEM((2, NUM_LEVELS, DC), jnp.bfloat16),
          pltpu.VMEM((2, S, DC), jnp.bfloat16),
          pltpu.VMEM((2, BPW, DC), jnp.bfloat16),
          pltpu.SemaphoreType.DMA,
          pltpu.SemaphoreType.DMA,
          pltpu.SemaphoreType.DMA,
          pltpu.SemaphoreType.DMA,
          pltpu.SemaphoreType.DMA,
          pltpu.SemaphoreType.DMA,
      ],
  )
  def enc_kernel(xt_hbm, id_hbm, lvl_hbm, out_hbm, xt_v, idx_v, lvl_v, id_v,
                 acc_v, s_lvl0, s_lvl1, s_id0, s_id1, s_out0, s_out1):
    wid = lax.axis_index("s") * 2 + lax.axis_index("c")
    s_lvl = [s_lvl0, s_lvl1]
    s_id = [s_id0, s_id1]
    s_out = [s_out0, s_out1]

    def start_in(ci, slot):
      col = ci * DC
      pltpu.async_copy(lvl_hbm.at[:, pl.ds(col, DC)], lvl_v.at[slot],
                       s_lvl[slot])
      pltpu.async_copy(id_hbm.at[:, pl.ds(col, DC)], id_v.at[slot],
                       s_id[slot])

    def wait_in(slot):
      pltpu.make_async_copy(lvl_hbm.at[:, pl.ds(0, DC)], lvl_v.at[slot],
                            s_lvl[slot]).wait()
      pltpu.make_async_copy(id_hbm.at[:, pl.ds(0, DC)], id_v.at[slot],
                            s_id[slot]).wait()

    def start_out(ci, slot):
      col = ci * DC
      pltpu.async_copy(acc_v.at[slot], out_hbm.at[wid, :, pl.ds(col, DC)],
                       s_out[slot])

    def wait_out(slot):
      pltpu.make_async_copy(acc_v.at[slot],
                            out_hbm.at[wid, :, pl.ds(0, DC)],
                            s_out[slot]).wait()

    start_in(0, 0)
    pltpu.sync_copy(xt_hbm.at[wid], xt_v)

    def idx_body(s, _):
      v = xt_v[s, :] * jnp.float32(NUM_LEVELS - 1) + jnp.float32(0.5)
      iv = v.astype(jnp.int32)
      iv = jnp.minimum(jnp.maximum(iv, 0), NUM_LEVELS - 1)
      idx_v[s, :] = iv
      return 0

    lax.fori_loop(0, S, idx_body, 0)

    def compute(slot, pi):
      lvl_s = lvl_v.at[slot]
      id_s = id_v.at[slot]
      acc_s = acc_v.at[slot]

      @pl.when(pi > 0)
      def _():
        wait_out(slot)

      zero = jnp.zeros((BLANES,), jnp.bfloat16)

      def zero_body(b, _):
        for t in range(TPB):
          acc_s[b, pl.ds(t * BLANES, BLANES)] = zero
        return 0

      lax.fori_loop(0, BPW, zero_body, 0)

      def g_core(s0, glen):
        vidx = [idx_v[s0 + g, :] for g in range(glen)]
        idrow = [
            [id_s[s0 + g, pl.ds(t * BLANES, BLANES)] for t in range(TPB)]
            for g in range(glen)
        ]
        for b in range(BPW):
          ib = [vidx[g][b] for g in range(glen)]
          accs = [acc_s[b, pl.ds(t * BLANES, BLANES)] for t in range(TPB)]
          for g in range(glen):
            lr = [lvl_s[ib[g], pl.ds(t * BLANES, BLANES)] for t in range(TPB)]
            for t in range(TPB):
              accs[t] = accs[t] + lr[t] * idrow[g][t]
          for t in range(TPB):
            acc_s[b, pl.ds(t * BLANES, BLANES)] = accs[t]

      def g_body(gi, _):
        g_core(gi * G, G)
        return 0

      lax.fori_loop(0, (S // G), g_body, 0)
      if S % G:
        g_core(S - S % G, S % G)

      one = jnp.float32(1.0).astype(jnp.bfloat16)
      mone = jnp.float32(-1.0).astype(jnp.bfloat16)
      zbf = jnp.float32(0.0).astype(jnp.bfloat16)

      def sign_body(b, _):
        for t in range(TPB):
          a = acc_s[b, pl.ds(t * BLANES, BLANES)]
          acc_s[b, pl.ds(t * BLANES, BLANES)] = jnp.where(a > zbf, one, mone)
        return 0

      lax.fori_loop(0, BPW, sign_body, 0)

    def pair_body(pi, _):
      c0 = pi * 2
      wait_in(0)
      start_in(c0 + 1, 1)
      compute(0, pi)
      start_out(c0, 0)
      wait_in(1)

      @pl.when(pi < NCHUNK // 2 - 1)
      def _():
        start_in(c0 + 2, 0)

      compute(1, pi)
      start_out(c0 + 1, 1)
      return 0

    lax.fori_loop(0, NCHUNK // 2, pair_body, 0)
    wait_out(0)
    wait_out(1)

  return enc_kernel(xt_pad, id_bf, lvl_bf)


def _tc_encode_classify(x_tc, id_bf, lvl_pad, wt_pad):
  cp = wt_pad.shape[1]

  def body(x_ref, id_ref, lvl_ref, w_ref, out_ref):
    idxf = x_ref[:] * jnp.float32(NUM_LEVELS - 1) + jnp.float32(0.5)
    idxi = jnp.clip(idxf.astype(jnp.int32), 0, NUM_LEVELS - 1)
    lcol = lax.broadcasted_iota(jnp.int32, (B_TC, LP), 1)
    lvl = lvl_ref[:]
    acc = jnp.zeros((B_TC, DT), jnp.float32)
    for s in range(S):
      oh = (idxi[:, s:s + 1] == lcol).astype(jnp.bfloat16)
      ts = lvl * id_ref[s:s + 1, :].astype(jnp.bfloat16)
      acc = acc + jnp.dot(oh, ts, preferred_element_type=jnp.float32)
    enc = jnp.where(acc > 0, jnp.float32(1.0), jnp.float32(-1.0))
    part = jnp.dot(enc, w_ref[:], preferred_element_type=jnp.float32)
    i = pl.program_id(0)

    @pl.when(i == 0)
    def _():
      out_ref[:] = part

    @pl.when(i != 0)
    def _():
      out_ref[:] = out_ref[:] + part

  return pl.pallas_call(
      body,
      grid=(D // DT,),
      in_specs=[
          pl.BlockSpec((B_TC, S), lambda i: (0, 0)),
          pl.BlockSpec((S, DT), lambda i: (0, i)),
          pl.BlockSpec((LP, DT), lambda i: (0, i)),
          pl.BlockSpec((DT, cp), lambda i: (i, 0)),
      ],
      out_specs=pl.BlockSpec((B_TC, cp), lambda i: (0, 0)),
      out_shape=jax.ShapeDtypeStruct((B_TC, cp), jnp.float32),
  )(x_tc, id_bf, lvl_pad, wt_pad)


def _tc_classify(enc, wt_pad):
  cp = wt_pad.shape[1]

  def body(enc_ref, w_ref, out_ref):
    out_ref[:] = jnp.dot(enc_ref[:].astype(jnp.float32), w_ref[:],
                         preferred_element_type=jnp.float32)

  return pl.pallas_call(
      body,
      out_shape=jax.ShapeDtypeStruct((enc.shape[0], cp), jnp.float32),
  )(enc, wt_pad)


@jax.jit
def kernel(x, id_weight, level_weight, classify_weight):
  id_bf = id_weight.astype(jnp.bfloat16)
  lvl_bf = level_weight.astype(jnp.bfloat16)
  xt_pad = jnp.zeros((SP, B_SC), jnp.float32).at[:S].set(x[:B_SC].T)
  xt3 = jnp.zeros((NW, SP, LANES), jnp.float32).at[:, :, :BPW].set(
      xt_pad.reshape(SP, NW, BPW).transpose(1, 0, 2))
  lvl_pad = jnp.zeros((LP, D), jnp.bfloat16).at[:NUM_LEVELS].set(lvl_bf)
  cp = 128
  wt_pad = jnp.zeros((D, cp), jnp.float32).at[:, :C].set(classify_weight.T)
  enc_sc = _sc_encode(xt3, id_bf, lvl_bf).reshape(B_SC, D)
  logit_tc = _tc_encode_classify(x[B_SC:], id_bf, lvl_pad, wt_pad)
  logit_sc = _tc_classify(enc_sc, wt_pad)
  return jnp.concatenate([logit_sc[:, :C], logit_tc[:, :C]], axis=0)
